# Initial kernel scaffold; baseline (speedup 1.0000x reference)
import jax
import jax.numpy as jnp
from jax.experimental import pallas as pl


def kernel(inputs, table):
    raise NotImplementedError("write your pallas kernel here")


# speedup vs baseline: 1.1028x; 1.1028x over previous
import functools

import jax
import jax.numpy as jnp
from jax import lax
from jax.experimental import pallas as pl
from jax.experimental.pallas import tpu as pltpu
from jax.experimental.pallas import tpu_sc as plsc

BATCH = 16384
HIST = 50
EMBED = 32
TOTAL = BATCH * HIST
NC = 2
NS = 16
NW = NC * NS
PER_W = TOTAL // NW
IDX_W = 128
ROWS_W = PER_W // IDX_W
CHUNK = 1024
G = CHUNK // IDX_W
N_CHUNKS = PER_W // CHUNK


def _build():
    mesh = plsc.VectorSubcoreMesh(core_axis_name="c", subcore_axis_name="s")

    @functools.partial(
        pl.kernel,
        mesh=mesh,
        out_type=jax.ShapeDtypeStruct((TOTAL, EMBED), jnp.float32),
        scratch_types=[
            pltpu.VMEM((ROWS_W, IDX_W), jnp.int32),
            pltpu.VMEM((CHUNK, EMBED), jnp.float32),
            pltpu.SemaphoreType.DMA,
        ],
        compiler_params=pltpu.CompilerParams(use_tc_tiling_on_sc=False),
    )
    def gather_kernel(idx_hbm, table_hbm, out_hbm, idx_v, rows_v, gsem):
        wid = lax.axis_index("s") * NC + lax.axis_index("c")
        base = wid * PER_W
        pltpu.sync_copy(idx_hbm.at[pl.ds(wid * ROWS_W, ROWS_W)], idx_v)

        @pl.loop(0, N_CHUNKS)
        def chunk_body(c):
            copies = []
            for g in range(G):
                copies.append(
                    pltpu.async_copy(
                        table_hbm.at[idx_v.at[c * G + g]],
                        rows_v.at[pl.ds(g * IDX_W, IDX_W)],
                        gsem,
                    )
                )
            for cp in copies:
                cp.wait()
            pltpu.sync_copy(rows_v, out_hbm.at[pl.ds(base + c * CHUNK, CHUNK)])

    return gather_kernel


_GATHER = _build()


def kernel(inputs, table):
    idx = inputs.reshape(TOTAL).astype(jnp.int32).reshape(TOTAL // IDX_W, IDX_W)
    out = _GATHER(idx, table)
    return out.reshape(BATCH, HIST, EMBED)


# speedup vs baseline: 1.6422x; 1.4891x over previous
import functools

import jax
import jax.numpy as jnp
from jax import lax
from jax.experimental import pallas as pl
from jax.experimental.pallas import tpu as pltpu
from jax.experimental.pallas import tpu_sc as plsc

BATCH = 16384
HIST = 50
EMBED = 32
NC = 2
NS = 16
NW = NC * NS
BT = BATCH // 128
UNITS = HIST * BT
PER_W = UNITS // NW
U = 5
NG = PER_W // U
L = 16


def _build():
    mesh = plsc.VectorSubcoreMesh(core_axis_name="c", subcore_axis_name="s")

    @functools.partial(
        pl.kernel,
        mesh=mesh,
        out_type=jax.ShapeDtypeStruct((HIST, EMBED // 8, BT, 8, 128), jnp.float32),
        scratch_types=[
            pltpu.VMEM((PER_W, 128), jnp.int32),
            pltpu.VMEM((U * 128, EMBED), jnp.float32),
            pltpu.VMEM((U * 128, EMBED), jnp.float32),
            pltpu.VMEM((U, EMBED, 128), jnp.float32),
            pltpu.VMEM((U, EMBED, 128), jnp.float32),
            pltpu.SemaphoreType.DMA,
            pltpu.SemaphoreType.DMA,
            pltpu.SemaphoreType.DMA,
            pltpu.SemaphoreType.DMA,
        ],
        compiler_params=pltpu.CompilerParams(
            use_tc_tiling_on_sc=False, needs_layout_passes=False),
    )
    def gather_kernel(idx_hbm, table_hbm, out_hbm, idx_v, rows_a, rows_b,
                      tb_a, tb_b, gsem_a, gsem_b, wsem_a, wsem_b):
        wid = lax.axis_index("s") * NC + lax.axis_index("c")
        ubase = wid * PER_W
        pltpu.sync_copy(idx_hbm.at[pl.ds(ubase, PER_W)], idx_v)

        rows_bufs = (rows_a, rows_b)
        tb_bufs = (tb_a, tb_b)
        gsems = (gsem_a, gsem_b)
        wsems = (wsem_a, wsem_b)

        def gather_copies(g, par):
            return [
                pltpu.make_async_copy(
                    table_hbm.at[idx_v.at[g * U + j]],
                    rows_bufs[par].at[pl.ds(j * 128, 128)],
                    gsems[par],
                )
                for j in range(U)
            ]

        def wb_copies(g, par):
            cps = []
            for j in range(U):
                u = ubase + g * U + j
                h = u // BT
                bt = lax.rem(u, BT)
                tb = tb_bufs[par]
                for et in range(EMBED // 8):
                    cps.append(
                        pltpu.make_async_copy(
                            tb.at[j, pl.ds(et * 8, 8)],
                            out_hbm.at[h, et, bt],
                            wsems[par],
                        )
                    )
            return cps

        def transpose_group(par):
            rows = rows_bufs[par]
            tb = tb_bufs[par]
            iot = lax.iota(jnp.int32, L)
            for j in range(U):

                @pl.loop(0, EMBED)
                def e_loop(e):
                    ce = jnp.broadcast_to(e, (L,))
                    for rb in range(128 // L):
                        ridx = j * 128 + rb * L + iot
                        x = plsc.load_gather(rows, [ridx, ce])
                        tb[j, e, pl.ds(rb * L, L)] = x

        for cp in gather_copies(0, 0):
            cp.start()

        @pl.loop(0, NG, step=2)
        def pair_body(g0):
            for p in range(2):
                g = g0 + p

                @pl.when(g + 1 < NG)
                def _fire_next():
                    for cp in gather_copies(g + 1, 1 - p):
                        cp.start()

                for cp in gather_copies(g, p):
                    cp.wait()

                @pl.when(g >= 2)
                def _drain_wb():
                    for cp in wb_copies(g - 2, p):
                        cp.wait()

                transpose_group(p)
                for cp in wb_copies(g, p):
                    cp.start()

        for cp in wb_copies(NG - 2, 0):
            cp.wait()
        for cp in wb_copies(NG - 1, 1):
            cp.wait()

    return gather_kernel


_GATHER = _build()


def kernel(inputs, table):
    idx_t = jnp.transpose(inputs.astype(jnp.int32)).reshape(UNITS, 128)
    packed = _GATHER(idx_t, table)
    return packed.transpose(2, 4, 0, 1, 3).reshape(BATCH, HIST, EMBED)
